# Initial kernel scaffold; baseline (speedup 1.0000x reference)
import jax
import jax.numpy as jnp
from jax.experimental import pallas as pl


def kernel(pointcloud, params):
    raise NotImplementedError("write your pallas kernel here")


# speedup vs baseline: 3.9612x; 3.9612x over previous
import functools

import jax
import jax.numpy as jnp
from jax.experimental import pallas as pl
from jax.experimental.pallas import tpu as pltpu

_SA1_SPECS = [(0.05, 16, [9, 16, 16, 32]), (0.1, 32, [9, 32, 32, 64])]
_SA2_SPECS = [(0.1, 16, [99, 64, 64, 128]), (0.2, 32, [99, 64, 96, 128])]
_NPOINT1, _NPOINT2 = 1024, 256


def _cumsum_lanes(x, n):
    s = 1
    while s < n:
        x = x + jnp.pad(x, ((0, 0), (s, 0)))[:, :n]
        s *= 2
    return x


def _fps_kernel(xs_ref, ys_ref, zs_ref, cx_ref, cy_ref, cz_ref, dist_ref,
                *, npoint, n):
    b = xs_ref.shape[0]
    xs = xs_ref[...]
    ys = ys_ref[...]
    zs = zs_ref[...]
    col = jax.lax.broadcasted_iota(jnp.int32, (b, n), 1)
    colp = jax.lax.broadcasted_iota(jnp.int32, (b, npoint), 1)
    dist_ref[...] = jnp.full((b, n), 1e10, jnp.float32)
    cx_ref[...] = jnp.zeros((b, npoint), jnp.float32)
    cy_ref[...] = jnp.zeros((b, npoint), jnp.float32)
    cz_ref[...] = jnp.zeros((b, npoint), jnp.float32)

    def body(t, far):
        sel = col == far
        cx = jnp.sum(jnp.where(sel, xs, 0.0), axis=1, keepdims=True)
        cy = jnp.sum(jnp.where(sel, ys, 0.0), axis=1, keepdims=True)
        cz = jnp.sum(jnp.where(sel, zs, 0.0), axis=1, keepdims=True)
        hit = colp == t
        cx_ref[...] = jnp.where(hit, cx, cx_ref[...])
        cy_ref[...] = jnp.where(hit, cy, cy_ref[...])
        cz_ref[...] = jnp.where(hit, cz, cz_ref[...])
        dx = xs - cx
        dy = ys - cy
        dz = zs - cz
        d = dx * dx + dy * dy + dz * dz
        dist = jnp.minimum(dist_ref[...], d)
        dist_ref[...] = dist
        mx = jnp.max(dist, axis=1, keepdims=True)
        far_new = jnp.min(jnp.where(dist == mx, col, n), axis=1, keepdims=True)
        return far_new

    jax.lax.fori_loop(0, npoint, body, jnp.zeros((b, 1), jnp.int32))


def _fps_centers(xs, ys, zs, npoint):
    b, n = xs.shape
    out_shape = [jax.ShapeDtypeStruct((b, npoint), jnp.float32)] * 3
    return pl.pallas_call(
        functools.partial(_fps_kernel, npoint=npoint, n=n),
        out_shape=out_shape,
        scratch_shapes=[pltpu.VMEM((b, n), jnp.float32)],
    )(xs, ys, zs)


def _group_kernel(xs_ref, ys_ref, zs_ref, pts_ref, cx_ref, cy_ref, cz_ref,
                  out_ref, *, r2, ns, rb, n, c, chunk):
    xs = xs_ref[0]
    ys = ys_ref[0]
    zs = zs_ref[0]
    cxb = cx_ref[0]
    cyb = cy_ref[0]
    czb = cz_ref[0]
    dx = cxb - xs
    dy = cyb - ys
    dz = czb - zs
    sqr = dx * dx + dy * dy + dz * dz
    mask = sqr <= r2
    rank = _cumsum_lanes(mask.astype(jnp.int32), n)
    count = rank[:, n - 1:n].reshape(rb, 1, 1)
    kio = jax.lax.broadcasted_iota(jnp.int32, (rb, ns, 1), 1)
    tgt = jnp.where(kio < count, kio + 1, 1)
    rank3 = rank.reshape(rb, 1, n)
    mask3 = mask.reshape(rb, 1, n)
    acc = jnp.zeros((rb * ns, c), jnp.float32)
    for j0 in range(0, n, chunk):
        sel = jnp.logical_and(rank3[:, :, j0:j0 + chunk] == tgt,
                              mask3[:, :, j0:j0 + chunk])
        self_f = sel.astype(jnp.float32).reshape(rb * ns, chunk)
        acc = acc + jnp.dot(self_f, pts_ref[0, j0:j0 + chunk, :],
                            preferred_element_type=jnp.float32,
                            precision=jax.lax.Precision.HIGHEST)
    acc3 = acc.reshape(rb, ns, c)
    chio = jax.lax.broadcasted_iota(jnp.int32, (rb, ns, c), 2)
    ctr = (jnp.where(chio == 0, cxb.reshape(rb, 1, 1), 0.0)
           + jnp.where(chio == 1, cyb.reshape(rb, 1, 1), 0.0)
           + jnp.where(chio == 2, czb.reshape(rb, 1, 1), 0.0))
    out_ref[0] = (acc3 - ctr).reshape(rb * ns, c)


def _group(xs3, ys3, zs3, pts, cx3, cy3, cz3, radius, ns, rb, chunk):
    b, _, n = xs3.shape
    c = pts.shape[-1]
    s = cx3.shape[1]
    r2 = radius * radius
    kern = functools.partial(_group_kernel, r2=r2, ns=ns, rb=rb, n=n, c=c,
                             chunk=chunk)
    row_spec = pl.BlockSpec((1, 1, n), lambda bi, si: (bi, 0, 0))
    ctr_spec = pl.BlockSpec((1, rb, 1), lambda bi, si: (bi, si, 0))
    return pl.pallas_call(
        kern,
        grid=(b, s // rb),
        in_specs=[row_spec, row_spec, row_spec,
                  pl.BlockSpec((1, n, c), lambda bi, si: (bi, 0, 0)),
                  ctr_spec, ctr_spec, ctr_spec],
        out_specs=pl.BlockSpec((1, rb * ns, c), lambda bi, si: (bi, si, 0)),
        out_shape=jax.ShapeDtypeStruct((b, s * ns, c), jnp.float32),
    )(xs3, ys3, zs3, pts, cx3, cy3, cz3)


def _mm_kernel(x_ref, w_ref, b_ref, y_ref, s1_ref, s2_ref):
    y = jnp.dot(x_ref[...], w_ref[...],
                preferred_element_type=jnp.float32) + b_ref[...]
    y_ref[...] = y
    p1 = jnp.sum(y, axis=0, keepdims=True)
    p2 = jnp.sum(y * y, axis=0, keepdims=True)

    @pl.when(pl.program_id(0) == 0)
    def _init():
        s1_ref[...] = p1
        s2_ref[...] = p2

    @pl.when(pl.program_id(0) > 0)
    def _acc():
        s1_ref[...] += p1
        s2_ref[...] += p2


def _mmbn_kernel(x_ref, s1i_ref, s2i_ref, g_ref, be_ref, w_ref, b_ref,
                 y_ref, s1_ref, s2_ref, *, inv_cnt):
    m = s1i_ref[...] * inv_cnt
    v = s2i_ref[...] * inv_cnt - m * m
    a = jnp.maximum(
        g_ref[...] * (x_ref[...] - m) / jnp.sqrt(v + 1e-5) + be_ref[...], 0.0)
    y = jnp.dot(a, w_ref[...], preferred_element_type=jnp.float32) + b_ref[...]
    y_ref[...] = y
    p1 = jnp.sum(y, axis=0, keepdims=True)
    p2 = jnp.sum(y * y, axis=0, keepdims=True)

    @pl.when(pl.program_id(0) == 0)
    def _init():
        s1_ref[...] = p1
        s2_ref[...] = p2

    @pl.when(pl.program_id(0) > 0)
    def _acc():
        s1_ref[...] += p1
        s2_ref[...] += p2


def _stats_out(r, cout, rblk):
    specs = [pl.BlockSpec((rblk, cout), lambda i: (i, 0)),
             pl.BlockSpec((1, cout), lambda i: (0, 0)),
             pl.BlockSpec((1, cout), lambda i: (0, 0))]
    shapes = [jax.ShapeDtypeStruct((r, cout), jnp.float32),
              jax.ShapeDtypeStruct((1, cout), jnp.float32),
              jax.ShapeDtypeStruct((1, cout), jnp.float32)]
    return specs, shapes


def _mm(x, w, b, rblk):
    r, cin = x.shape
    cout = w.shape[1]
    out_specs, out_shape = _stats_out(r, cout, rblk)
    return pl.pallas_call(
        _mm_kernel,
        grid=(r // rblk,),
        in_specs=[pl.BlockSpec((rblk, cin), lambda i: (i, 0)),
                  pl.BlockSpec((cin, cout), lambda i: (0, 0)),
                  pl.BlockSpec((1, cout), lambda i: (0, 0))],
        out_specs=out_specs,
        out_shape=out_shape,
    )(x, w, b.reshape(1, cout))


def _mmbn(x, s1, s2, g, be, w, b, inv_cnt, rblk):
    r, cin = x.shape
    cout = w.shape[1]
    out_specs, out_shape = _stats_out(r, cout, rblk)
    stat_spec = pl.BlockSpec((1, cin), lambda i: (0, 0))
    return pl.pallas_call(
        functools.partial(_mmbn_kernel, inv_cnt=inv_cnt),
        grid=(r // rblk,),
        in_specs=[pl.BlockSpec((rblk, cin), lambda i: (i, 0)),
                  stat_spec, stat_spec, stat_spec, stat_spec,
                  pl.BlockSpec((cin, cout), lambda i: (0, 0)),
                  pl.BlockSpec((1, cout), lambda i: (0, 0))],
        out_specs=out_specs,
        out_shape=out_shape,
    )(x, s1, s2, g.reshape(1, cin), be.reshape(1, cin), w, b.reshape(1, cout))


def _pool_kernel(y_ref, s1_ref, s2_ref, g_ref, be_ref, o_ref, *, inv_cnt):
    d = y_ref.shape[2]
    m = (s1_ref[...] * inv_cnt).reshape(1, 1, d)
    v = (s2_ref[...] * inv_cnt).reshape(1, 1, d) - m * m
    g = g_ref[...].reshape(1, 1, d)
    be = be_ref[...].reshape(1, 1, d)
    a = jnp.maximum(g * (y_ref[...] - m) / jnp.sqrt(v + 1e-5) + be, 0.0)
    o_ref[...] = jnp.max(a, axis=1)


def _pool(y3, s1, s2, g, be, inv_cnt, gb):
    rows, ns, d = y3.shape
    stat_spec = pl.BlockSpec((1, d), lambda i: (0, 0))
    return pl.pallas_call(
        functools.partial(_pool_kernel, inv_cnt=inv_cnt),
        grid=(rows // gb,),
        in_specs=[pl.BlockSpec((gb, ns, d), lambda i: (i, 0, 0)),
                  stat_spec, stat_spec, stat_spec, stat_spec],
        out_specs=pl.BlockSpec((gb, d), lambda i: (i, 0)),
        out_shape=jax.ShapeDtypeStruct((rows, d), jnp.float32),
    )(y3, s1, s2, g.reshape(1, d), be.reshape(1, d))


def _sa_msg(xs, ys, zs, pts, npoint, specs, params, rb, chunk):
    b, n = xs.shape
    cx, cy, cz = _fps_centers(xs, ys, zs, npoint)
    xs3 = xs.reshape(b, 1, n)
    ys3 = ys.reshape(b, 1, n)
    zs3 = zs.reshape(b, 1, n)
    cx3 = cx.reshape(b, npoint, 1)
    cy3 = cy.reshape(b, npoint, 1)
    cz3 = cz.reshape(b, npoint, 1)
    outs = []
    for (radius, ns, dims), mlp in zip(specs, params):
        grouped = _group(xs3, ys3, zs3, pts, cx3, cy3, cz3, radius, ns, rb,
                         chunk)
        x = grouped.reshape(b * npoint * ns, dims[0])
        inv_cnt = 1.0 / (b * npoint * ns)
        w, bb, g, be = mlp[0]
        y, s1, s2 = _mm(x, w, bb, rblk=512)
        for w2, b2, g2, be2 in mlp[1:]:
            y, s1n, s2n = _mmbn(y, s1, s2, g, be, w2, b2, inv_cnt, rblk=512)
            s1, s2, g, be = s1n, s2n, g2, be2
        pooled = _pool(y.reshape(b * npoint, ns, dims[-1]), s1, s2, g, be,
                       inv_cnt, gb=128)
        outs.append(pooled.reshape(b, npoint, dims[-1]))
    return (cx, cy, cz), jnp.concatenate(outs, axis=-1)


def kernel(pointcloud, params):
    b, n, _ = pointcloud.shape
    xs = pointcloud[..., 0]
    ys = pointcloud[..., 1]
    zs = pointcloud[..., 2]
    (cx1, cy1, cz1), f1 = _sa_msg(xs, ys, zs, pointcloud, _NPOINT1,
                                  _SA1_SPECS, params["sa1"], rb=8, chunk=2048)
    pts2 = jnp.concatenate([jnp.stack([cx1, cy1, cz1], axis=-1), f1], axis=-1)
    (cx2, cy2, cz2), f2 = _sa_msg(cx1, cy1, cz1, pts2, _NPOINT2,
                                  _SA2_SPECS, params["sa2"], rb=8, chunk=1024)
    lin, _, _ = _mm(f2.reshape(b * _NPOINT2, f2.shape[-1]),
                    params["linear_w"], params["linear_b"], rblk=512)
    xyz2 = jnp.stack([cx2, cy2, cz2], axis=-1)
    return jnp.concatenate([xyz2, lin.reshape(b, _NPOINT2, -1)], axis=-1)


# speedup vs baseline: 4.2658x; 1.0769x over previous
import functools

import jax
import jax.numpy as jnp
from jax.experimental import pallas as pl
from jax.experimental.pallas import tpu as pltpu

_SA1_SPECS = [(0.05, 16, [9, 16, 16, 32]), (0.1, 32, [9, 32, 32, 64])]
_SA2_SPECS = [(0.1, 16, [99, 64, 64, 128]), (0.2, 32, [99, 64, 96, 128])]
_NPOINT1, _NPOINT2 = 1024, 256


def _cumsum_lanes(x, n):
    s = 1
    while s < n:
        x = x + jnp.pad(x, ((0, 0), (s, 0)))[:, :n]
        s *= 2
    return x


def _fps_kernel(xs_ref, ys_ref, zs_ref, cx_ref, cy_ref, cz_ref, dist_ref,
                *, npoint, n):
    b = xs_ref.shape[0]
    xs = xs_ref[...]
    ys = ys_ref[...]
    zs = zs_ref[...]
    col = jax.lax.broadcasted_iota(jnp.int32, (b, n), 1)
    colp = jax.lax.broadcasted_iota(jnp.int32, (b, npoint), 1)
    dist_ref[...] = jnp.full((b, n), 1e10, jnp.float32)
    cx_ref[...] = jnp.zeros((b, npoint), jnp.float32)
    cy_ref[...] = jnp.zeros((b, npoint), jnp.float32)
    cz_ref[...] = jnp.zeros((b, npoint), jnp.float32)

    def body(t, far):
        sel = col == far
        cx = jnp.sum(jnp.where(sel, xs, 0.0), axis=1, keepdims=True)
        cy = jnp.sum(jnp.where(sel, ys, 0.0), axis=1, keepdims=True)
        cz = jnp.sum(jnp.where(sel, zs, 0.0), axis=1, keepdims=True)
        hit = colp == t
        cx_ref[...] = jnp.where(hit, cx, cx_ref[...])
        cy_ref[...] = jnp.where(hit, cy, cy_ref[...])
        cz_ref[...] = jnp.where(hit, cz, cz_ref[...])
        dx = xs - cx
        dy = ys - cy
        dz = zs - cz
        d = dx * dx + dy * dy + dz * dz
        dist = jnp.minimum(dist_ref[...], d)
        dist_ref[...] = dist
        mx = jnp.max(dist, axis=1, keepdims=True)
        far_new = jnp.min(jnp.where(dist == mx, col, n), axis=1, keepdims=True)
        return far_new

    jax.lax.fori_loop(0, npoint, body, jnp.zeros((b, 1), jnp.int32))


def _fps_centers(xs, ys, zs, npoint):
    b, n = xs.shape
    out_shape = [jax.ShapeDtypeStruct((b, npoint), jnp.float32)] * 3
    return pl.pallas_call(
        functools.partial(_fps_kernel, npoint=npoint, n=n),
        out_shape=out_shape,
        scratch_shapes=[pltpu.VMEM((b, n), jnp.float32)],
    )(xs, ys, zs)


def _group_kernel(xs_ref, ys_ref, zs_ref, pts_ref, cx_ref, cy_ref, cz_ref,
                  out_ref, *, r2, ns, rb, n, c, chunk):
    xs = xs_ref[0]
    ys = ys_ref[0]
    zs = zs_ref[0]
    cxb = cx_ref[0]
    cyb = cy_ref[0]
    czb = cz_ref[0]
    dx = cxb - xs
    dy = cyb - ys
    dz = czb - zs
    sqr = dx * dx + dy * dy + dz * dz
    mask = sqr <= r2
    rank = _cumsum_lanes(mask.astype(jnp.int32), n)
    count = rank[:, n - 1:n].reshape(rb, 1, 1)
    kio = jax.lax.broadcasted_iota(jnp.int32, (rb, ns, 1), 1)
    tgt = jnp.where(kio < count, kio + 1, 1)
    rankm3 = jnp.where(mask, rank, 0).reshape(rb, 1, n)
    acc = jnp.zeros((rb * ns, c), jnp.float32)
    for j0 in range(0, n, chunk):
        sel = rankm3[:, :, j0:j0 + chunk] == tgt
        self_f = sel.astype(jnp.float32).reshape(rb * ns, chunk)
        acc = acc + jnp.dot(self_f, pts_ref[0, j0:j0 + chunk, :],
                            preferred_element_type=jnp.float32,
                            precision=jax.lax.Precision.HIGHEST)
    acc3 = acc.reshape(rb, ns, c)
    chio = jax.lax.broadcasted_iota(jnp.int32, (rb, ns, c), 2)
    ctr = (jnp.where(chio == 0, cxb.reshape(rb, 1, 1), 0.0)
           + jnp.where(chio == 1, cyb.reshape(rb, 1, 1), 0.0)
           + jnp.where(chio == 2, czb.reshape(rb, 1, 1), 0.0))
    out_ref[0] = (acc3 - ctr).reshape(rb * ns, c)


def _group(xs3, ys3, zs3, pts, cx3, cy3, cz3, radius, ns, rb, chunk):
    b, _, n = xs3.shape
    c = pts.shape[-1]
    s = cx3.shape[1]
    r2 = radius * radius
    kern = functools.partial(_group_kernel, r2=r2, ns=ns, rb=rb, n=n, c=c,
                             chunk=chunk)
    row_spec = pl.BlockSpec((1, 1, n), lambda bi, si: (bi, 0, 0))
    ctr_spec = pl.BlockSpec((1, rb, 1), lambda bi, si: (bi, si, 0))
    return pl.pallas_call(
        kern,
        grid=(b, s // rb),
        in_specs=[row_spec, row_spec, row_spec,
                  pl.BlockSpec((1, n, c), lambda bi, si: (bi, 0, 0)),
                  ctr_spec, ctr_spec, ctr_spec],
        out_specs=pl.BlockSpec((1, rb * ns, c), lambda bi, si: (bi, si, 0)),
        out_shape=jax.ShapeDtypeStruct((b, s * ns, c), jnp.float32),
        compiler_params=pltpu.CompilerParams(
            dimension_semantics=("parallel", "parallel")),
    )(xs3, ys3, zs3, pts, cx3, cy3, cz3)


def _mm_kernel(x_ref, w_ref, b_ref, y_ref, s1_ref, s2_ref):
    y = jnp.dot(x_ref[...], w_ref[...],
                preferred_element_type=jnp.float32) + b_ref[...]
    y_ref[...] = y
    p1 = jnp.sum(y, axis=0, keepdims=True)
    p2 = jnp.sum(y * y, axis=0, keepdims=True)

    @pl.when(pl.program_id(0) == 0)
    def _init():
        s1_ref[...] = p1
        s2_ref[...] = p2

    @pl.when(pl.program_id(0) > 0)
    def _acc():
        s1_ref[...] += p1
        s2_ref[...] += p2


def _mmbn_kernel(x_ref, s1i_ref, s2i_ref, g_ref, be_ref, w_ref, b_ref,
                 y_ref, s1_ref, s2_ref, *, inv_cnt):
    m = s1i_ref[...] * inv_cnt
    v = s2i_ref[...] * inv_cnt - m * m
    a = jnp.maximum(
        g_ref[...] * (x_ref[...] - m) / jnp.sqrt(v + 1e-5) + be_ref[...], 0.0)
    y = jnp.dot(a, w_ref[...], preferred_element_type=jnp.float32) + b_ref[...]
    y_ref[...] = y
    p1 = jnp.sum(y, axis=0, keepdims=True)
    p2 = jnp.sum(y * y, axis=0, keepdims=True)

    @pl.when(pl.program_id(0) == 0)
    def _init():
        s1_ref[...] = p1
        s2_ref[...] = p2

    @pl.when(pl.program_id(0) > 0)
    def _acc():
        s1_ref[...] += p1
        s2_ref[...] += p2


def _stats_out(r, cout, rblk):
    specs = [pl.BlockSpec((rblk, cout), lambda i: (i, 0)),
             pl.BlockSpec((1, cout), lambda i: (0, 0)),
             pl.BlockSpec((1, cout), lambda i: (0, 0))]
    shapes = [jax.ShapeDtypeStruct((r, cout), jnp.float32),
              jax.ShapeDtypeStruct((1, cout), jnp.float32),
              jax.ShapeDtypeStruct((1, cout), jnp.float32)]
    return specs, shapes


def _mm(x, w, b, rblk):
    r, cin = x.shape
    cout = w.shape[1]
    out_specs, out_shape = _stats_out(r, cout, rblk)
    return pl.pallas_call(
        _mm_kernel,
        grid=(r // rblk,),
        in_specs=[pl.BlockSpec((rblk, cin), lambda i: (i, 0)),
                  pl.BlockSpec((cin, cout), lambda i: (0, 0)),
                  pl.BlockSpec((1, cout), lambda i: (0, 0))],
        out_specs=out_specs,
        out_shape=out_shape,
    )(x, w, b.reshape(1, cout))


def _mmbn(x, s1, s2, g, be, w, b, inv_cnt, rblk):
    r, cin = x.shape
    cout = w.shape[1]
    out_specs, out_shape = _stats_out(r, cout, rblk)
    stat_spec = pl.BlockSpec((1, cin), lambda i: (0, 0))
    return pl.pallas_call(
        functools.partial(_mmbn_kernel, inv_cnt=inv_cnt),
        grid=(r // rblk,),
        in_specs=[pl.BlockSpec((rblk, cin), lambda i: (i, 0)),
                  stat_spec, stat_spec, stat_spec, stat_spec,
                  pl.BlockSpec((cin, cout), lambda i: (0, 0)),
                  pl.BlockSpec((1, cout), lambda i: (0, 0))],
        out_specs=out_specs,
        out_shape=out_shape,
    )(x, s1, s2, g.reshape(1, cin), be.reshape(1, cin), w, b.reshape(1, cout))


def _pool_kernel(y_ref, s1_ref, s2_ref, g_ref, be_ref, o_ref, *, inv_cnt):
    d = y_ref.shape[2]
    m = (s1_ref[...] * inv_cnt).reshape(1, 1, d)
    v = (s2_ref[...] * inv_cnt).reshape(1, 1, d) - m * m
    g = g_ref[...].reshape(1, 1, d)
    be = be_ref[...].reshape(1, 1, d)
    a = jnp.maximum(g * (y_ref[...] - m) / jnp.sqrt(v + 1e-5) + be, 0.0)
    o_ref[...] = jnp.max(a, axis=1)


def _pool(y3, s1, s2, g, be, inv_cnt, gb):
    rows, ns, d = y3.shape
    stat_spec = pl.BlockSpec((1, d), lambda i: (0, 0))
    return pl.pallas_call(
        functools.partial(_pool_kernel, inv_cnt=inv_cnt),
        grid=(rows // gb,),
        in_specs=[pl.BlockSpec((gb, ns, d), lambda i: (i, 0, 0)),
                  stat_spec, stat_spec, stat_spec, stat_spec],
        out_specs=pl.BlockSpec((gb, d), lambda i: (i, 0)),
        out_shape=jax.ShapeDtypeStruct((rows, d), jnp.float32),
        compiler_params=pltpu.CompilerParams(
            dimension_semantics=("parallel",)),
    )(y3, s1, s2, g.reshape(1, d), be.reshape(1, d))


def _sa_msg(xs, ys, zs, pts, npoint, specs, params, rb, chunk):
    b, n = xs.shape
    cx, cy, cz = _fps_centers(xs, ys, zs, npoint)
    xs3 = xs.reshape(b, 1, n)
    ys3 = ys.reshape(b, 1, n)
    zs3 = zs.reshape(b, 1, n)
    cx3 = cx.reshape(b, npoint, 1)
    cy3 = cy.reshape(b, npoint, 1)
    cz3 = cz.reshape(b, npoint, 1)
    outs = []
    for (radius, ns, dims), mlp in zip(specs, params):
        grouped = _group(xs3, ys3, zs3, pts, cx3, cy3, cz3, radius, ns, rb,
                         chunk)
        x = grouped.reshape(b * npoint * ns, dims[0])
        inv_cnt = 1.0 / (b * npoint * ns)
        w, bb, g, be = mlp[0]
        y, s1, s2 = _mm(x, w, bb, rblk=512)
        for w2, b2, g2, be2 in mlp[1:]:
            y, s1n, s2n = _mmbn(y, s1, s2, g, be, w2, b2, inv_cnt, rblk=512)
            s1, s2, g, be = s1n, s2n, g2, be2
        pooled = _pool(y.reshape(b * npoint, ns, dims[-1]), s1, s2, g, be,
                       inv_cnt, gb=128)
        outs.append(pooled.reshape(b, npoint, dims[-1]))
    return (cx, cy, cz), jnp.concatenate(outs, axis=-1)


def kernel(pointcloud, params):
    b, n, _ = pointcloud.shape
    xs = pointcloud[..., 0]
    ys = pointcloud[..., 1]
    zs = pointcloud[..., 2]
    (cx1, cy1, cz1), f1 = _sa_msg(xs, ys, zs, pointcloud, _NPOINT1,
                                  _SA1_SPECS, params["sa1"], rb=8, chunk=2048)
    pts2 = jnp.concatenate([jnp.stack([cx1, cy1, cz1], axis=-1), f1], axis=-1)
    (cx2, cy2, cz2), f2 = _sa_msg(cx1, cy1, cz1, pts2, _NPOINT2,
                                  _SA2_SPECS, params["sa2"], rb=8, chunk=1024)
    lin, _, _ = _mm(f2.reshape(b * _NPOINT2, f2.shape[-1]),
                    params["linear_w"], params["linear_b"], rblk=512)
    xyz2 = jnp.stack([cx2, cy2, cz2], axis=-1)
    return jnp.concatenate([xyz2, lin.reshape(b, _NPOINT2, -1)], axis=-1)
